# baseline (speedup 1.0000x reference)
import functools

import jax
import jax.numpy as jnp
from jax import lax
from jax.experimental import pallas as pl
from jax.experimental.pallas import tpu as pltpu
from jax.experimental.pallas import tpu_sc as plsc

NC = 2
NS = 16
NW = NC * NS

CHUNK = 128


def _make_gather(n_total: int, dim: int):
    assert n_total % NW == 0
    per_w = n_total // NW
    assert per_w % CHUNK == 0
    n_chunks = per_w // CHUNK

    mesh = plsc.VectorSubcoreMesh(
        core_axis_name="c", subcore_axis_name="s",
        num_cores=NC, num_subcores=NS)

    @functools.partial(
        pl.kernel,
        out_type=jax.ShapeDtypeStruct((n_total, dim), jnp.float32),
        mesh=mesh,
        scratch_types=[
            pltpu.VMEM((n_chunks, CHUNK), jnp.int32),
            pltpu.VMEM((CHUNK, dim), jnp.float32),
            pltpu.SemaphoreType.DMA,
        ],
        compiler_params=pltpu.CompilerParams(use_tc_tiling_on_sc=False),
    )
    def gather_kernel(vocab_hbm, idx_hbm, out_hbm, idx_v, rows_v, sem):
        wid = lax.axis_index("s") * NC + lax.axis_index("c")
        base = wid * per_w
        pltpu.sync_copy(idx_hbm.at[pl.ds(wid * n_chunks, n_chunks)], idx_v)

        def body(j, carry):
            pltpu.async_copy(vocab_hbm.at[idx_v.at[j]], rows_v, sem).wait()
            pltpu.sync_copy(rows_v, out_hbm.at[pl.ds(base + j * CHUNK, CHUNK)])
            return carry

        lax.fori_loop(0, n_chunks, body, 0, unroll=False)

    return gather_kernel


def kernel(s, vocab):
    b, t = s.shape
    dim = vocab.shape[1]
    idx = s.reshape(-1, CHUNK).astype(jnp.int32)
    out = _make_gather(b * t, dim)(vocab, idx)
    return out.reshape(b, t, dim)


# speedup vs baseline: 1.1138x; 1.1138x over previous
import functools

import jax
import jax.numpy as jnp
from jax import lax
from jax.experimental import pallas as pl
from jax.experimental.pallas import tpu as pltpu
from jax.experimental.pallas import tpu_sc as plsc

NC = 2
NS = 16
NW = NC * NS

CHUNK = 128
NBUF = 8


def _make_gather(n_total: int, dim: int):
    assert n_total % (NW * CHUNK) == 0
    per_w = n_total // NW
    n_chunks = per_w // CHUNK
    assert n_chunks % NBUF == 0
    n_groups = n_chunks // NBUF

    mesh = plsc.VectorSubcoreMesh(
        core_axis_name="c", subcore_axis_name="s",
        num_cores=NC, num_subcores=NS)

    @functools.partial(
        pl.kernel,
        out_type=jax.ShapeDtypeStruct((n_total, dim), jnp.float32),
        mesh=mesh,
        scratch_types=[
            pltpu.VMEM((n_chunks, CHUNK), jnp.int32),
            pltpu.VMEM((NBUF, CHUNK, dim), jnp.float32),
            pltpu.SemaphoreType.DMA((NBUF,)),
            pltpu.SemaphoreType.DMA((NBUF,)),
        ],
        compiler_params=pltpu.CompilerParams(use_tc_tiling_on_sc=False),
    )
    def gather_kernel(vocab_hbm, idx_hbm, out_hbm, idx_v, rows_v, gsem, wsem):
        wid = lax.axis_index("s") * NC + lax.axis_index("c")
        base = wid * per_w
        pltpu.sync_copy(idx_hbm.at[pl.ds(wid * n_chunks, n_chunks)], idx_v)

        def start_gather(k, b):
            pltpu.async_copy(vocab_hbm.at[idx_v.at[k]], rows_v.at[b],
                             gsem.at[b])

        def start_write(k, b):
            pltpu.async_copy(rows_v.at[b],
                             out_hbm.at[pl.ds(base + k * CHUNK, CHUNK)],
                             wsem.at[b])

        def wait_gather(b):
            pltpu.make_async_copy(vocab_hbm.at[idx_v.at[0]], rows_v.at[b],
                                  gsem.at[b]).wait()

        def wait_write(b):
            pltpu.make_async_copy(rows_v.at[b],
                                  out_hbm.at[pl.ds(base, CHUNK)],
                                  wsem.at[b]).wait()

        for b in range(NBUF):
            start_gather(b, b)

        def body(g, carry):
            for b in range(NBUF):
                wait_gather(b)
                start_write(g * NBUF + b, b)
            for b in range(NBUF):
                wait_write(b)
                start_gather((g + 1) * NBUF + b, b)
            return carry

        lax.fori_loop(0, n_groups - 1, body, 0, unroll=False)

        last = (n_groups - 1) * NBUF
        for b in range(NBUF):
            wait_gather(b)
            start_write(last + b, b)
        for b in range(NBUF):
            wait_write(b)

    return gather_kernel


def kernel(s, vocab):
    b, t = s.shape
    dim = vocab.shape[1]
    idx = s.reshape(-1, CHUNK).astype(jnp.int32)
    out = _make_gather(b * t, dim)(vocab, idx)
    return out.reshape(b, t, dim)
